# baseline (speedup 1.0000x reference)
import functools

import jax
import jax.numpy as jnp
import numpy as np
from jax import lax
from jax.experimental import pallas as pl
from jax.experimental.pallas import tpu as pltpu
from jax.experimental.pallas import tpu_sc as plsc

_EMB_LIST = [100, 11, 11, 11, 9, 4, 9, 5, 4, 8]
_TOTAL = 172
_NFEAT = 10
_OFFSETS = [int(x) for x in np.concatenate([[0], np.cumsum(_EMB_LIST)[:-1]])]

_N = 100000
_K_SC = 40000
_TC_BLOCK = 15000

_CHUNK = 160
_NGROUPS = _CHUNK // 16
_NSPANS = _TOTAL // 16
_TAILSPAN = _TOTAL - _NSPANS * 16
_NW = 32


_FEAT_OF_COL = np.repeat(np.arange(_NFEAT), _EMB_LIST)
_LOCAL_OF_COL = np.arange(_TOTAL) - np.asarray(_OFFSETS)[_FEAT_OF_COL]
_FEAT_MAP = (np.arange(_NFEAT)[:, None] == _FEAT_OF_COL[None, :]).astype(
    np.float32
)


def _tc_body(atom_ref, fmap_ref, local_ref, out_ref):
    a = atom_ref[...].astype(jnp.float32)
    g = jax.lax.dot_general(
        a,
        fmap_ref[...],
        (((1,), (0,)), ((), ())),
        preferred_element_type=jnp.float32,
    )
    out_ref[...] = jnp.where(g == local_ref[...], 1.0, 0.0)


def _tc_kernel(atom_part):
    n = atom_part.shape[0]
    grid = n // _TC_BLOCK
    fmap = jnp.asarray(_FEAT_MAP)
    local = jnp.asarray(_LOCAL_OF_COL, dtype=jnp.float32)[None, :]
    return pl.pallas_call(
        _tc_body,
        out_shape=jax.ShapeDtypeStruct((n, _TOTAL), jnp.float32),
        grid=(grid,),
        in_specs=[
            pl.BlockSpec((_TC_BLOCK, _NFEAT), lambda i: (i, 0)),
            pl.BlockSpec((_NFEAT, _TOTAL), lambda i: (0, 0)),
            pl.BlockSpec((1, _TOTAL), lambda i: (0, 0)),
        ],
        out_specs=pl.BlockSpec((_TC_BLOCK, _TOTAL), lambda i: (i, 0)),
    )(atom_part, fmap, local)


def _sc_body(atom_hbm, out_hbm, atom_v0, atom_v1, out_v0, out_v1,
             idx_v0, idx_v1, sem_i0, sem_i1, sem_o0, sem_o1):
    wid = lax.axis_index("s") * 2 + lax.axis_index("c")
    lanes = lax.broadcasted_iota(jnp.int32, (16,), 0)
    ones = jnp.full((16,), 1.0, dtype=jnp.float32)
    zeros = jnp.zeros((16,), dtype=jnp.float32)
    tail_mask = lanes < _TAILSPAN

    atom_bufs = (atom_v0, atom_v1)
    out_bufs = (out_v0, out_v1)
    idx_bufs = (idx_v0, idx_v1)
    in_sems = (sem_i0, sem_i1)
    out_sems = (sem_o0, sem_o1)

    nchunks = _K_SC // _CHUNK
    n_mine = (nchunks - 1 - wid) // _NW + 1

    def row0(j):
        return (wid + j * _NW) * _CHUNK

    def zero_buf(out_v):
        def zrow(r, c):
            for k in range(_NSPANS):
                out_v[r, pl.ds(k * 16, 16)] = zeros
            plsc.store_scatter(
                out_v,
                [jnp.full((16,), r, dtype=jnp.int32), _NSPANS * 16 + lanes],
                zeros,
                mask=tail_mask,
            )
            return c

        lax.fori_loop(0, _CHUNK, zrow, 0)

    def scatter_buf(atom_v, out_v, idx_v):
        def group_body(g, c):
            rows = g * 16 + lanes
            for i in range(_NFEAT):
                vals = plsc.load_gather(
                    atom_v, [rows, jnp.full((16,), i, dtype=jnp.int32)]
                )
                mask = (vals >= 0) & (vals < _EMB_LIST[i])
                cols = jnp.where(mask, vals + _OFFSETS[i], 0)
                plsc.store_scatter(out_v, [rows, cols], ones, mask=mask)
                idx_v[pl.ds((g * _NFEAT + i) * 16, 16)] = cols
            return c

        lax.fori_loop(0, _NGROUPS, group_body, 0)

    def unscatter_buf(out_v, idx_v):
        def group_body(g, c):
            rows = g * 16 + lanes
            for i in range(_NFEAT):
                cols = idx_v[pl.ds((g * _NFEAT + i) * 16, 16)]
                plsc.store_scatter(out_v, [rows, cols], zeros)
            return c

        lax.fori_loop(0, _NGROUPS, group_body, 0)

    zero_buf(out_bufs[0])
    zero_buf(out_bufs[1])

    @pl.when(n_mine > 0)
    def _():
        pltpu.async_copy(
            atom_hbm.at[pl.ds(row0(0), _CHUNK), :], atom_bufs[0], in_sems[0]
        )

    npairs = (n_mine + 1) // 2

    def pair_body(j2, carry):
        for b in range(2):
            j = j2 * 2 + b

            @pl.when(j < n_mine)
            def _():
                r0 = row0(j)

                @pl.when(j >= 2)
                def _():
                    pltpu.make_async_copy(
                        out_bufs[b],
                        out_hbm.at[pl.ds(r0, _CHUNK), :],
                        out_sems[b],
                    ).wait()
                    unscatter_buf(out_bufs[b], idx_bufs[b])

                pltpu.make_async_copy(
                    atom_hbm.at[pl.ds(r0, _CHUNK), :],
                    atom_bufs[b],
                    in_sems[b],
                ).wait()
                scatter_buf(atom_bufs[b], out_bufs[b], idx_bufs[b])
                pltpu.async_copy(
                    out_bufs[b],
                    out_hbm.at[pl.ds(r0, _CHUNK), :],
                    out_sems[b],
                )

                @pl.when(j + 1 < n_mine)
                def _():
                    pltpu.async_copy(
                        atom_hbm.at[pl.ds(row0(j + 1), _CHUNK), :],
                        atom_bufs[1 - b],
                        in_sems[1 - b],
                    )

        return carry

    lax.fori_loop(0, npairs, pair_body, 0)

    for b in range(2):

        @pl.when(n_mine > b)
        def _():
            last_j = jnp.where((n_mine - 1) % 2 == b, n_mine - 1, n_mine - 2)
            pltpu.make_async_copy(
                out_bufs[b],
                out_hbm.at[pl.ds(row0(last_j), _CHUNK), :],
                out_sems[b],
            ).wait()


def _sc_kernel(atom_part):
    mesh = plsc.VectorSubcoreMesh(core_axis_name="c", subcore_axis_name="s")
    run = pl.kernel(
        _sc_body,
        out_type=jax.ShapeDtypeStruct((_K_SC, _TOTAL), jnp.float32),
        mesh=mesh,
        scratch_types=[
            pltpu.VMEM((_CHUNK, _NFEAT), jnp.int32),
            pltpu.VMEM((_CHUNK, _NFEAT), jnp.int32),
            pltpu.VMEM((_CHUNK, _TOTAL), jnp.float32),
            pltpu.VMEM((_CHUNK, _TOTAL), jnp.float32),
            pltpu.VMEM((_CHUNK * _NFEAT,), jnp.int32),
            pltpu.VMEM((_CHUNK * _NFEAT,), jnp.int32),
            pltpu.SemaphoreType.DMA,
            pltpu.SemaphoreType.DMA,
            pltpu.SemaphoreType.DMA,
            pltpu.SemaphoreType.DMA,
        ],
        compiler_params=pltpu.CompilerParams(needs_layout_passes=False),
    )
    return run(atom_part)


@jax.jit
def kernel(atom):
    atom = atom.astype(jnp.int32)
    if _K_SC == 0:
        return _tc_kernel(atom)
    sc_out = _sc_kernel(atom[:_K_SC])
    tc_out = _tc_kernel(atom[_K_SC:])
    return jnp.concatenate([sc_out, tc_out], axis=0)
